# baseline (speedup 1.0000x reference)
import functools

import jax
import jax.numpy as jnp
from jax import lax
from jax.experimental import pallas as pl
from jax.experimental.pallas import tpu as pltpu
from jax.experimental.pallas import tpu_sc as plsc

_ROWS = 4096
_COLS = 8192
_SAMPLES = 32
_STRIDE = _COLS // _SAMPLES
_OFFSET = _STRIDE // 2
_TOTAL = _ROWS * _SAMPLES

_NUM_CORES = 2
_NUM_SUBCORES = 16
_NW = _NUM_CORES * _NUM_SUBCORES
_CHUNK = _TOTAL // _NW

_mesh = plsc.VectorSubcoreMesh(core_axis_name="c", subcore_axis_name="s")


@functools.partial(
    pl.kernel,
    mesh=_mesh,
    out_type=jax.ShapeDtypeStruct((_TOTAL, 1), jnp.float32),
    scratch_types=[
        pltpu.VMEM((_CHUNK, 1), jnp.float32),
    ],
    compiler_params=pltpu.CompilerParams(use_tc_tiling_on_sc=False),
)
def _sc_sample(x_hbm, out_hbm, vals_v):
    wid = lax.axis_index("s") * _NUM_CORES + lax.axis_index("c")
    base = wid * _CHUNK
    pltpu.sync_copy(x_hbm.at[pl.ds(base, _CHUNK), pl.ds(_OFFSET, 1)], vals_v)
    pltpu.sync_copy(vals_v, out_hbm.at[pl.ds(base, _CHUNK), :])


def kernel(x):
    x2 = x.reshape(_TOTAL, _STRIDE)
    out = _sc_sample(x2)
    return out.reshape(_ROWS, _SAMPLES)


# speedup vs baseline: 1.6424x; 1.6424x over previous
import functools

import jax
import jax.numpy as jnp
import numpy as np
from jax import lax
from jax.experimental import pallas as pl
from jax.experimental.pallas import tpu as pltpu
from jax.experimental.pallas import tpu_sc as plsc

_ROWS = 4096
_COLS = 8192
_SAMPLES = 32
_TOTAL = _ROWS * _SAMPLES

_NUM_CORES = 2
_NUM_SUBCORES = 16
_NW = _NUM_CORES * _NUM_SUBCORES
_CHUNK = _TOTAL // _NW

_FLAT_IDX = (256 * np.arange(_TOTAL, dtype=np.int64) + 128).astype(np.int32)

_mesh = plsc.VectorSubcoreMesh(core_axis_name="c", subcore_axis_name="s")


@functools.partial(
    pl.kernel,
    mesh=_mesh,
    out_type=jax.ShapeDtypeStruct((_TOTAL,), jnp.float32),
    scratch_types=[
        pltpu.VMEM((_CHUNK,), jnp.int32),
        pltpu.VMEM((_CHUNK,), jnp.float32),
        pltpu.SemaphoreType.DMA,
    ],
)
def _sc_gather(x_hbm, idx_hbm, out_hbm, idx_v, vals_v, sem):
    wid = lax.axis_index("s") * _NUM_CORES + lax.axis_index("c")
    base = wid * _CHUNK
    pltpu.sync_copy(idx_hbm.at[pl.ds(base, _CHUNK)], idx_v)
    pltpu.async_copy(x_hbm.at[idx_v], vals_v, sem).wait()
    pltpu.sync_copy(vals_v, out_hbm.at[pl.ds(base, _CHUNK)])


def kernel(x):
    flat = x.reshape(-1)
    idx = jnp.asarray(_FLAT_IDX)
    out = _sc_gather(flat, idx)
    return out.reshape(_ROWS, _SAMPLES)
